# baseline (speedup 1.0000x reference)
import functools

import jax
import jax.numpy as jnp
from jax import lax
from jax.experimental import pallas as pl
from jax.experimental.pallas import tpu as pltpu
from jax.experimental.pallas import tpu_sc as plsc

BATCH = 16384
MLP_EMB = 64
GMF_EMB = 32
H1 = 64

_NC = 2
_NS = 16
_NW = _NC * _NS
_BPW = BATCH // _NW
_CH = 128
_NCHUNK = _BPW // _CH


def _gather_body(users_hbm, items_hbm, mu_tab, mi_tab, gu_tab, gi_tab,
                 mu_out, mi_out, gu_out, gi_out,
                 uidx_v, iidx_v, mu_v, mi_v, gu_v, gi_v, sem):
    wid = lax.axis_index("s") * _NC + lax.axis_index("c")
    base = wid * _BPW
    pltpu.sync_copy(users_hbm.at[wid], uidx_v)
    pltpu.sync_copy(items_hbm.at[wid], iidx_v)
    copies = []
    for j in range(_NCHUNK):
        dst = pl.ds(j * _CH, _CH)
        copies.append(pltpu.async_copy(mu_tab.at[uidx_v.at[j]], mu_v.at[dst], sem))
        copies.append(pltpu.async_copy(mi_tab.at[iidx_v.at[j]], mi_v.at[dst], sem))
        copies.append(pltpu.async_copy(gu_tab.at[uidx_v.at[j]], gu_v.at[dst], sem))
        copies.append(pltpu.async_copy(gi_tab.at[iidx_v.at[j]], gi_v.at[dst], sem))
    for c in copies:
        c.wait()
    out_sl = pl.ds(base, _BPW)
    pltpu.sync_copy(mu_v, mu_out.at[out_sl])
    pltpu.sync_copy(mi_v, mi_out.at[out_sl])
    pltpu.sync_copy(gu_v, gu_out.at[out_sl])
    pltpu.sync_copy(gi_v, gi_out.at[out_sl])


_gather = functools.partial(
    pl.kernel,
    mesh=plsc.VectorSubcoreMesh(core_axis_name="c", subcore_axis_name="s"),
    out_type=[
        jax.ShapeDtypeStruct((BATCH, MLP_EMB), jnp.float32),
        jax.ShapeDtypeStruct((BATCH, MLP_EMB), jnp.float32),
        jax.ShapeDtypeStruct((BATCH, GMF_EMB), jnp.float32),
        jax.ShapeDtypeStruct((BATCH, GMF_EMB), jnp.float32),
    ],
    scratch_types=[
        pltpu.VMEM((_NCHUNK, _CH), jnp.int32),
        pltpu.VMEM((_NCHUNK, _CH), jnp.int32),
        pltpu.VMEM((_BPW, MLP_EMB), jnp.float32),
        pltpu.VMEM((_BPW, MLP_EMB), jnp.float32),
        pltpu.VMEM((_BPW, GMF_EMB), jnp.float32),
        pltpu.VMEM((_BPW, GMF_EMB), jnp.float32),
        pltpu.SemaphoreType.DMA,
    ],
    compiler_params=pltpu.CompilerParams(use_tc_tiling_on_sc=False),
)(_gather_body)


def _mlp_body(mu_ref, mi_ref, gu_ref, gi_ref, W1_ref, b1_ref, W2_ref, b2_ref,
              out_ref):
    w1 = W1_ref[...]
    h = jnp.dot(mu_ref[...], w1[:MLP_EMB, :], preferred_element_type=jnp.float32)
    h += jnp.dot(mi_ref[...], w1[MLP_EMB:, :], preferred_element_type=jnp.float32)
    h = jnp.maximum(h + b1_ref[...], 0.0)
    gmf = gu_ref[...] * gi_ref[...]
    w2 = W2_ref[...]
    z = jnp.dot(h, w2[:H1, :], preferred_element_type=jnp.float32)
    z += jnp.dot(gmf, w2[H1:, :], preferred_element_type=jnp.float32)
    out_ref[...] = jax.nn.sigmoid(z + b2_ref[...])


def _mlp(mu, mi, gu, gi, W1, b1, W2, b2, blk=2048):
    nblk = BATCH // blk
    return pl.pallas_call(
        _mlp_body,
        grid=(nblk,),
        in_specs=[
            pl.BlockSpec((blk, MLP_EMB), lambda i: (i, 0)),
            pl.BlockSpec((blk, MLP_EMB), lambda i: (i, 0)),
            pl.BlockSpec((blk, GMF_EMB), lambda i: (i, 0)),
            pl.BlockSpec((blk, GMF_EMB), lambda i: (i, 0)),
            pl.BlockSpec((2 * MLP_EMB, H1), lambda i: (0, 0)),
            pl.BlockSpec((1, H1), lambda i: (0, 0)),
            pl.BlockSpec((H1 + GMF_EMB, 1), lambda i: (0, 0)),
            pl.BlockSpec((1, 1), lambda i: (0, 0)),
        ],
        out_specs=pl.BlockSpec((blk, 1), lambda i: (i, 0)),
        out_shape=jax.ShapeDtypeStruct((BATCH, 1), jnp.float32),
    )(mu, mi, gu, gi, W1, b1, W2, b2)


@jax.jit
def kernel(users, items, mlp_user_emb, mlp_item_emb, gmf_user_emb,
           gmf_item_emb, W1, b1, W2, b2):
    users_r = users.reshape(_NW, _NCHUNK, _CH)
    items_r = items.reshape(_NW, _NCHUNK, _CH)
    mu, mi, gu, gi = _gather(users_r, items_r, mlp_user_emb, mlp_item_emb,
                             gmf_user_emb, gmf_item_emb)
    return _mlp(mu, mi, gu, gi, W1, b1.reshape(1, H1), W2, b2.reshape(1, 1))


# speedup vs baseline: 1.5109x; 1.5109x over previous
import functools

import jax
import jax.numpy as jnp
from jax import lax
from jax.experimental import pallas as pl
from jax.experimental.pallas import tpu as pltpu
from jax.experimental.pallas import tpu_sc as plsc

V = 1000000
BATCH = 16384
MLP_EMB = 64
GMF_EMB = 32
H1 = 64

_NC = 2
_NS = 16
_NW = _NC * _NS
_BPW = BATCH // _NW
_NCH = _BPW // 128
_RBLK = 2048
_RNB = (V + _RBLK - 1) // _RBLK


def _repack_body(mu_t_ref, gu_t_ref, out_ref):
    muT = mu_t_ref[...].T
    guT = gu_t_ref[...].T
    out_ref[...] = jnp.concatenate(
        [muT, guT, jnp.zeros((_RBLK, 32), jnp.float32)], axis=1)


def _repack(mu_t, gu_t):
    return pl.pallas_call(
        _repack_body,
        grid=(_RNB,),
        in_specs=[
            pl.BlockSpec((MLP_EMB, _RBLK), lambda i: (0, i)),
            pl.BlockSpec((GMF_EMB, _RBLK), lambda i: (0, i)),
        ],
        out_specs=pl.BlockSpec((_RBLK, 128), lambda i: (i, 0)),
        out_shape=jax.ShapeDtypeStruct((V, 128), jnp.float32),
    )(mu_t, gu_t)


def _gather_body(uidx_hbm, iidx_hbm, u_tab, i_tab, gu_out, gi_out,
                 uidx_v, iidx_v, rows_v, sem):
    wid = lax.axis_index("s") * _NC + lax.axis_index("c")
    base = wid * _BPW
    pltpu.sync_copy(uidx_hbm.at[wid], uidx_v)
    pltpu.sync_copy(iidx_hbm.at[wid], iidx_v)
    for j in range(_NCH):
        pltpu.async_copy(u_tab.at[uidx_v.at[j]],
                         rows_v.at[pl.ds(j * 128, 128)], sem)
    for j in range(_NCH):
        pltpu.make_async_copy(u_tab.at[uidx_v.at[j]],
                              rows_v.at[pl.ds(j * 128, 128)], sem).wait()
    pltpu.sync_copy(rows_v, gu_out.at[pl.ds(base, _BPW)])
    for j in range(_NCH):
        pltpu.async_copy(i_tab.at[iidx_v.at[j]],
                         rows_v.at[pl.ds(j * 128, 128)], sem)
    for j in range(_NCH):
        pltpu.make_async_copy(i_tab.at[iidx_v.at[j]],
                              rows_v.at[pl.ds(j * 128, 128)], sem).wait()
    pltpu.sync_copy(rows_v, gi_out.at[pl.ds(base, _BPW)])


_gather = functools.partial(
    pl.kernel,
    mesh=plsc.VectorSubcoreMesh(core_axis_name="c", subcore_axis_name="s"),
    out_type=[
        jax.ShapeDtypeStruct((BATCH, 128), jnp.float32),
        jax.ShapeDtypeStruct((BATCH, 128), jnp.float32),
    ],
    scratch_types=[
        pltpu.VMEM((_NCH, 128), jnp.int32),
        pltpu.VMEM((_NCH, 128), jnp.int32),
        pltpu.VMEM((_BPW, 128), jnp.float32),
        pltpu.SemaphoreType.DMA,
    ],
    compiler_params=pltpu.CompilerParams(use_tc_tiling_on_sc=True),
)(_gather_body)


def _mlp_body(gu_ref, gi_ref, W1_ref, b1_ref, W2_ref, b2_ref, out_ref):
    gu_all = gu_ref[...]
    gi_all = gi_ref[...]
    w1 = W1_ref[...]
    mu = gu_all[:, :MLP_EMB]
    mi = gi_all[:, :MLP_EMB]
    h = jnp.dot(mu, w1[:MLP_EMB, :], preferred_element_type=jnp.float32)
    h += jnp.dot(mi, w1[MLP_EMB:, :], preferred_element_type=jnp.float32)
    h = jnp.maximum(h + b1_ref[...], 0.0)
    gmf = (gu_all[:, MLP_EMB:MLP_EMB + GMF_EMB]
           * gi_all[:, MLP_EMB:MLP_EMB + GMF_EMB])
    w2 = W2_ref[...]
    z = jnp.dot(h, w2[:H1, :], preferred_element_type=jnp.float32)
    z += jnp.dot(gmf, w2[H1:, :], preferred_element_type=jnp.float32)
    out_ref[...] = jax.nn.sigmoid(z + b2_ref[...])


def _mlp(GU, GI, W1, b1, W2, b2, blk=2048):
    nblk = BATCH // blk
    return pl.pallas_call(
        _mlp_body,
        grid=(nblk,),
        in_specs=[
            pl.BlockSpec((blk, 128), lambda i: (i, 0)),
            pl.BlockSpec((blk, 128), lambda i: (i, 0)),
            pl.BlockSpec((2 * MLP_EMB, H1), lambda i: (0, 0)),
            pl.BlockSpec((1, H1), lambda i: (0, 0)),
            pl.BlockSpec((H1 + GMF_EMB, 1), lambda i: (0, 0)),
            pl.BlockSpec((1, 1), lambda i: (0, 0)),
        ],
        out_specs=pl.BlockSpec((blk, 1), lambda i: (i, 0)),
        out_shape=jax.ShapeDtypeStruct((BATCH, 1), jnp.float32),
    )(GU, GI, W1, b1, W2, b2)


@jax.jit
def kernel(users, items, mlp_user_emb, mlp_item_emb, gmf_user_emb,
           gmf_item_emb, W1, b1, W2, b2):
    U = _repack(mlp_user_emb.T, gmf_user_emb.T)
    I = _repack(mlp_item_emb.T, gmf_item_emb.T)
    GU, GI = _gather(users.reshape(_NW, _NCH, 128),
                     items.reshape(_NW, _NCH, 128), U, I)
    return _mlp(GU, GI, W1, b1.reshape(1, H1), W2, b2.reshape(1, 1))


# speedup vs baseline: 1.8966x; 1.2553x over previous
import functools

import jax
import jax.numpy as jnp
from jax import lax
from jax.experimental import pallas as pl
from jax.experimental.pallas import tpu as pltpu
from jax.experimental.pallas import tpu_sc as plsc

V = 1000000
BATCH = 16384
MLP_EMB = 64
GMF_EMB = 32
H1 = 64

_NC = 2
_NS = 16
_NW = _NC * _NS
_BPW = BATCH // _NW
_NCH = _BPW // 128
_RBLK = 4096
_RNB = (V + _RBLK - 1) // _RBLK

_DN = (((0,), (0,)), ((), ()))


def _repack_body(mu_t_ref, gu_t_ref, w1h_ref, out_ref):
    hT = lax.dot_general(mu_t_ref[...], w1h_ref[...], _DN,
                         preferred_element_type=jnp.float32)
    eye32 = (lax.broadcasted_iota(jnp.int32, (GMF_EMB, GMF_EMB), 0)
             == lax.broadcasted_iota(jnp.int32, (GMF_EMB, GMF_EMB), 1)
             ).astype(jnp.float32)
    gT = lax.dot_general(gu_t_ref[...], eye32, _DN,
                         preferred_element_type=jnp.float32)
    out_ref[:, :MLP_EMB] = hT
    out_ref[:, MLP_EMB:MLP_EMB + GMF_EMB] = gT


def _repack(mu_t, gu_t, w1_half):
    return pl.pallas_call(
        _repack_body,
        grid=(_RNB,),
        in_specs=[
            pl.BlockSpec((MLP_EMB, _RBLK), lambda i: (0, i)),
            pl.BlockSpec((GMF_EMB, _RBLK), lambda i: (0, i)),
            pl.BlockSpec((MLP_EMB, H1), lambda i: (0, 0)),
        ],
        out_specs=pl.BlockSpec((_RBLK, 128), lambda i: (i, 0)),
        out_shape=jax.ShapeDtypeStruct((V, 128), jnp.float32),
    )(mu_t, gu_t, w1_half)


def _gather_body(uidx_hbm, iidx_hbm, u_tab, i_tab, gu_out, gi_out,
                 uidx_v, iidx_v, rows_v, sem):
    wid = lax.axis_index("s") * _NC + lax.axis_index("c")
    base = wid * _BPW
    pltpu.sync_copy(uidx_hbm.at[wid], uidx_v)
    pltpu.sync_copy(iidx_hbm.at[wid], iidx_v)
    for j in range(_NCH):
        pltpu.async_copy(u_tab.at[uidx_v.at[j]],
                         rows_v.at[pl.ds(j * 128, 128)], sem)
    for j in range(_NCH):
        pltpu.make_async_copy(u_tab.at[uidx_v.at[j]],
                              rows_v.at[pl.ds(j * 128, 128)], sem).wait()
    pltpu.sync_copy(rows_v, gu_out.at[pl.ds(base, _BPW)])
    for j in range(_NCH):
        pltpu.async_copy(i_tab.at[iidx_v.at[j]],
                         rows_v.at[pl.ds(j * 128, 128)], sem)
    for j in range(_NCH):
        pltpu.make_async_copy(i_tab.at[iidx_v.at[j]],
                              rows_v.at[pl.ds(j * 128, 128)], sem).wait()
    pltpu.sync_copy(rows_v, gi_out.at[pl.ds(base, _BPW)])


_gather = functools.partial(
    pl.kernel,
    mesh=plsc.VectorSubcoreMesh(core_axis_name="c", subcore_axis_name="s"),
    out_type=[
        jax.ShapeDtypeStruct((BATCH, 128), jnp.float32),
        jax.ShapeDtypeStruct((BATCH, 128), jnp.float32),
    ],
    scratch_types=[
        pltpu.VMEM((_NCH, 128), jnp.int32),
        pltpu.VMEM((_NCH, 128), jnp.int32),
        pltpu.VMEM((_BPW, 128), jnp.float32),
        pltpu.SemaphoreType.DMA,
    ],
    compiler_params=pltpu.CompilerParams(use_tc_tiling_on_sc=True),
)(_gather_body)


def _mlp_body(gu_ref, gi_ref, b1_ref, W2_ref, b2_ref, out_ref):
    gu_all = gu_ref[...]
    gi_all = gi_ref[...]
    h = jnp.maximum(
        gu_all[:, :H1] + gi_all[:, :H1] + b1_ref[...], 0.0)
    gmf = (gu_all[:, MLP_EMB:MLP_EMB + GMF_EMB]
           * gi_all[:, MLP_EMB:MLP_EMB + GMF_EMB])
    w2 = W2_ref[...]
    z = jnp.dot(h, w2[:H1, :], preferred_element_type=jnp.float32)
    z += jnp.dot(gmf, w2[H1:, :], preferred_element_type=jnp.float32)
    out_ref[...] = jax.nn.sigmoid(z + b2_ref[...])


def _mlp(GU, GI, b1, W2, b2, blk=2048):
    nblk = BATCH // blk
    return pl.pallas_call(
        _mlp_body,
        grid=(nblk,),
        in_specs=[
            pl.BlockSpec((blk, 128), lambda i: (i, 0)),
            pl.BlockSpec((blk, 128), lambda i: (i, 0)),
            pl.BlockSpec((1, H1), lambda i: (0, 0)),
            pl.BlockSpec((H1 + GMF_EMB, 1), lambda i: (0, 0)),
            pl.BlockSpec((1, 1), lambda i: (0, 0)),
        ],
        out_specs=pl.BlockSpec((blk, 1), lambda i: (i, 0)),
        out_shape=jax.ShapeDtypeStruct((BATCH, 1), jnp.float32),
    )(GU, GI, b1, W2, b2)


@jax.jit
def kernel(users, items, mlp_user_emb, mlp_item_emb, gmf_user_emb,
           gmf_item_emb, W1, b1, W2, b2):
    U = _repack(mlp_user_emb.T, gmf_user_emb.T, W1[:MLP_EMB])
    I = _repack(mlp_item_emb.T, gmf_item_emb.T, W1[MLP_EMB:])
    GU, GI = _gather(users.reshape(_NW, _NCH, 128),
                     items.reshape(_NW, _NCH, 128), U, I)
    return _mlp(GU, GI, b1.reshape(1, H1), W2, b2.reshape(1, 1))


# speedup vs baseline: 2.5389x; 1.3387x over previous
import functools

import jax
import jax.numpy as jnp
from jax import lax
from jax.experimental import pallas as pl
from jax.experimental.pallas import tpu as pltpu
from jax.experimental.pallas import tpu_sc as plsc

V = 1000000
BATCH = 16384
MLP_EMB = 64
GMF_EMB = 32
H1 = 64

_NC = 2
_NS = 16
_NW = _NC * _NS
_BPW = BATCH // _NW
_NCH = _BPW // 128
_RBLK = 4096
_RNB = (V + _RBLK - 1) // _RBLK

_DN = (((0,), (0,)), ((), ()))


def _eye(n):
    return (lax.broadcasted_iota(jnp.int32, (n, n), 0)
            == lax.broadcasted_iota(jnp.int32, (n, n), 1)
            ).astype(jnp.bfloat16)


def _repack_body(mu_t_ref, gu_t_ref, mi_t_ref, gi_t_ref, w1_ref,
                 u_out_ref, i_out_ref):
    w1 = w1_ref[...].astype(jnp.bfloat16)
    eye32 = _eye(GMF_EMB)
    for tref, wmat, oref in (
        (mu_t_ref, w1[:MLP_EMB], u_out_ref),
        (mi_t_ref, w1[MLP_EMB:], i_out_ref),
    ):
        oref[:, :MLP_EMB] = lax.dot_general(
            tref[...].astype(jnp.bfloat16), wmat, _DN,
            preferred_element_type=jnp.float32)
    for tref, oref in ((gu_t_ref, u_out_ref), (gi_t_ref, i_out_ref)):
        oref[:, MLP_EMB:MLP_EMB + GMF_EMB] = lax.dot_general(
            tref[...].astype(jnp.bfloat16), eye32, _DN,
            preferred_element_type=jnp.float32)


def _repack(mu_t, gu_t, mi_t, gi_t, W1):
    return pl.pallas_call(
        _repack_body,
        grid=(_RNB,),
        in_specs=[
            pl.BlockSpec((MLP_EMB, _RBLK), lambda i: (0, i)),
            pl.BlockSpec((GMF_EMB, _RBLK), lambda i: (0, i)),
            pl.BlockSpec((MLP_EMB, _RBLK), lambda i: (0, i)),
            pl.BlockSpec((GMF_EMB, _RBLK), lambda i: (0, i)),
            pl.BlockSpec((2 * MLP_EMB, H1), lambda i: (0, 0)),
        ],
        out_specs=[
            pl.BlockSpec((_RBLK, 128), lambda i: (i, 0)),
            pl.BlockSpec((_RBLK, 128), lambda i: (i, 0)),
        ],
        out_shape=[
            jax.ShapeDtypeStruct((V, 128), jnp.float32),
            jax.ShapeDtypeStruct((V, 128), jnp.float32),
        ],
    )(mu_t, gu_t, mi_t, gi_t, W1)


def _gather_body(uidx_hbm, iidx_hbm, u_tab, i_tab, gu_out, gi_out,
                 uidx_v, iidx_v, rows_v, sem):
    wid = lax.axis_index("s") * _NC + lax.axis_index("c")
    base = wid * _BPW
    pltpu.sync_copy(uidx_hbm.at[wid], uidx_v)
    pltpu.sync_copy(iidx_hbm.at[wid], iidx_v)
    for j in range(_NCH):
        pltpu.async_copy(u_tab.at[uidx_v.at[j]],
                         rows_v.at[pl.ds(j * 128, 128)], sem)
    for j in range(_NCH):
        pltpu.make_async_copy(u_tab.at[uidx_v.at[j]],
                              rows_v.at[pl.ds(j * 128, 128)], sem).wait()
    pltpu.sync_copy(rows_v, gu_out.at[pl.ds(base, _BPW)])
    for j in range(_NCH):
        pltpu.async_copy(i_tab.at[iidx_v.at[j]],
                         rows_v.at[pl.ds(j * 128, 128)], sem)
    for j in range(_NCH):
        pltpu.make_async_copy(i_tab.at[iidx_v.at[j]],
                              rows_v.at[pl.ds(j * 128, 128)], sem).wait()
    pltpu.sync_copy(rows_v, gi_out.at[pl.ds(base, _BPW)])


_gather = functools.partial(
    pl.kernel,
    mesh=plsc.VectorSubcoreMesh(core_axis_name="c", subcore_axis_name="s"),
    out_type=[
        jax.ShapeDtypeStruct((BATCH, 128), jnp.float32),
        jax.ShapeDtypeStruct((BATCH, 128), jnp.float32),
    ],
    scratch_types=[
        pltpu.VMEM((_NCH, 128), jnp.int32),
        pltpu.VMEM((_NCH, 128), jnp.int32),
        pltpu.VMEM((_BPW, 128), jnp.float32),
        pltpu.SemaphoreType.DMA,
    ],
    compiler_params=pltpu.CompilerParams(use_tc_tiling_on_sc=True),
)(_gather_body)


def _mlp_body(gu_ref, gi_ref, b1_ref, W2_ref, b2_ref, out_ref):
    gu_all = gu_ref[...]
    gi_all = gi_ref[...]
    h = jnp.maximum(
        gu_all[:, :H1] + gi_all[:, :H1] + b1_ref[...], 0.0)
    gmf = (gu_all[:, MLP_EMB:MLP_EMB + GMF_EMB]
           * gi_all[:, MLP_EMB:MLP_EMB + GMF_EMB])
    w2 = W2_ref[...]
    z = jnp.dot(h, w2[:H1, :], preferred_element_type=jnp.float32)
    z += jnp.dot(gmf, w2[H1:, :], preferred_element_type=jnp.float32)
    out_ref[...] = jax.nn.sigmoid(z + b2_ref[...])


def _mlp(GU, GI, b1, W2, b2, blk=2048):
    nblk = BATCH // blk
    return pl.pallas_call(
        _mlp_body,
        grid=(nblk,),
        in_specs=[
            pl.BlockSpec((blk, 128), lambda i: (i, 0)),
            pl.BlockSpec((blk, 128), lambda i: (i, 0)),
            pl.BlockSpec((1, H1), lambda i: (0, 0)),
            pl.BlockSpec((H1 + GMF_EMB, 1), lambda i: (0, 0)),
            pl.BlockSpec((1, 1), lambda i: (0, 0)),
        ],
        out_specs=pl.BlockSpec((blk, 1), lambda i: (i, 0)),
        out_shape=jax.ShapeDtypeStruct((BATCH, 1), jnp.float32),
    )(GU, GI, b1, W2, b2)


@jax.jit
def kernel(users, items, mlp_user_emb, mlp_item_emb, gmf_user_emb,
           gmf_item_emb, W1, b1, W2, b2):
    U, I = _repack(mlp_user_emb.T, gmf_user_emb.T,
                   mlp_item_emb.T, gmf_item_emb.T, W1)
    GU, GI = _gather(users.reshape(_NW, _NCH, 128),
                     items.reshape(_NW, _NCH, 128), U, I)
    return _mlp(GU, GI, b1.reshape(1, H1), W2, b2.reshape(1, 1))


# speedup vs baseline: 2.8274x; 1.1136x over previous
import functools

import jax
import jax.numpy as jnp
from jax import lax
from jax.experimental import pallas as pl
from jax.experimental.pallas import tpu as pltpu
from jax.experimental.pallas import tpu_sc as plsc

V = 1000000
BATCH = 16384
MLP_EMB = 64
GMF_EMB = 32
H1 = 64

_NC = 2
_NS = 16
_NW = _NC * _NS
_BPW = BATCH // _NW
_NCH = _BPW // 128
_RBLK = 8192
_RNB = (V + _RBLK - 1) // _RBLK

_DN = (((0,), (0,)), ((), ()))


def _eye(n):
    return (lax.broadcasted_iota(jnp.int32, (n, n), 0)
            == lax.broadcasted_iota(jnp.int32, (n, n), 1)
            ).astype(jnp.bfloat16)


def _repack_body(mu_t_ref, gu_t_ref, mi_t_ref, gi_t_ref, w1_ref,
                 u_out_ref, i_out_ref):
    w1 = w1_ref[...].astype(jnp.bfloat16)
    eye32 = _eye(GMF_EMB)
    for tref, wmat, oref in (
        (mu_t_ref, w1[:MLP_EMB], u_out_ref),
        (mi_t_ref, w1[MLP_EMB:], i_out_ref),
    ):
        oref[:, :MLP_EMB] = lax.dot_general(
            tref[...].astype(jnp.bfloat16), wmat, _DN,
            preferred_element_type=jnp.float32)
    for tref, oref in ((gu_t_ref, u_out_ref), (gi_t_ref, i_out_ref)):
        oref[:, MLP_EMB:MLP_EMB + GMF_EMB] = lax.dot_general(
            tref[...].astype(jnp.bfloat16), eye32, _DN,
            preferred_element_type=jnp.float32)


def _repack(mu_t, gu_t, mi_t, gi_t, W1):
    return pl.pallas_call(
        _repack_body,
        grid=(_RNB,),
        in_specs=[
            pl.BlockSpec((MLP_EMB, _RBLK), lambda i: (0, i)),
            pl.BlockSpec((GMF_EMB, _RBLK), lambda i: (0, i)),
            pl.BlockSpec((MLP_EMB, _RBLK), lambda i: (0, i)),
            pl.BlockSpec((GMF_EMB, _RBLK), lambda i: (0, i)),
            pl.BlockSpec((2 * MLP_EMB, H1), lambda i: (0, 0)),
        ],
        out_specs=[
            pl.BlockSpec((_RBLK, 128), lambda i: (i, 0)),
            pl.BlockSpec((_RBLK, 128), lambda i: (i, 0)),
        ],
        out_shape=[
            jax.ShapeDtypeStruct((V, 128), jnp.float32),
            jax.ShapeDtypeStruct((V, 128), jnp.float32),
        ],
    )(mu_t, gu_t, mi_t, gi_t, W1)


def _gather_body(uidx_hbm, iidx_hbm, u_tab, i_tab, gu_out, gi_out,
                 uidx_v, iidx_v, rows_v, sem):
    wid = lax.axis_index("s") * _NC + lax.axis_index("c")
    base = wid * _BPW
    pltpu.sync_copy(uidx_hbm.at[wid], uidx_v)
    pltpu.sync_copy(iidx_hbm.at[wid], iidx_v)
    for j in range(_NCH):
        pltpu.async_copy(u_tab.at[uidx_v.at[j]],
                         rows_v.at[pl.ds(j * 128, 128)], sem)
    for j in range(_NCH):
        pltpu.make_async_copy(u_tab.at[uidx_v.at[j]],
                              rows_v.at[pl.ds(j * 128, 128)], sem).wait()
    pltpu.sync_copy(rows_v, gu_out.at[pl.ds(base, _BPW)])
    for j in range(_NCH):
        pltpu.async_copy(i_tab.at[iidx_v.at[j]],
                         rows_v.at[pl.ds(j * 128, 128)], sem)
    for j in range(_NCH):
        pltpu.make_async_copy(i_tab.at[iidx_v.at[j]],
                              rows_v.at[pl.ds(j * 128, 128)], sem).wait()
    pltpu.sync_copy(rows_v, gi_out.at[pl.ds(base, _BPW)])


_gather = functools.partial(
    pl.kernel,
    mesh=plsc.VectorSubcoreMesh(core_axis_name="c", subcore_axis_name="s"),
    out_type=[
        jax.ShapeDtypeStruct((BATCH, 128), jnp.float32),
        jax.ShapeDtypeStruct((BATCH, 128), jnp.float32),
    ],
    scratch_types=[
        pltpu.VMEM((_NCH, 128), jnp.int32),
        pltpu.VMEM((_NCH, 128), jnp.int32),
        pltpu.VMEM((_BPW, 128), jnp.float32),
        pltpu.SemaphoreType.DMA,
    ],
    compiler_params=pltpu.CompilerParams(use_tc_tiling_on_sc=True),
)(_gather_body)


def _mlp_body(gu_ref, gi_ref, b1_ref, W2_ref, b2_ref, out_ref):
    gu_all = gu_ref[...]
    gi_all = gi_ref[...]
    h = jnp.maximum(
        gu_all[:, :H1] + gi_all[:, :H1] + b1_ref[...], 0.0)
    gmf = (gu_all[:, MLP_EMB:MLP_EMB + GMF_EMB]
           * gi_all[:, MLP_EMB:MLP_EMB + GMF_EMB])
    w2 = W2_ref[...]
    z = jnp.dot(h, w2[:H1, :], preferred_element_type=jnp.float32)
    z += jnp.dot(gmf, w2[H1:, :], preferred_element_type=jnp.float32)
    out_ref[...] = jax.nn.sigmoid(z + b2_ref[...])


def _mlp(GU, GI, b1, W2, b2, blk=2048):
    nblk = BATCH // blk
    return pl.pallas_call(
        _mlp_body,
        grid=(nblk,),
        in_specs=[
            pl.BlockSpec((blk, 128), lambda i: (i, 0)),
            pl.BlockSpec((blk, 128), lambda i: (i, 0)),
            pl.BlockSpec((1, H1), lambda i: (0, 0)),
            pl.BlockSpec((H1 + GMF_EMB, 1), lambda i: (0, 0)),
            pl.BlockSpec((1, 1), lambda i: (0, 0)),
        ],
        out_specs=pl.BlockSpec((blk, 1), lambda i: (i, 0)),
        out_shape=jax.ShapeDtypeStruct((BATCH, 1), jnp.float32),
    )(GU, GI, b1, W2, b2)


@jax.jit
def kernel(users, items, mlp_user_emb, mlp_item_emb, gmf_user_emb,
           gmf_item_emb, W1, b1, W2, b2):
    U, I = _repack(mlp_user_emb.T, gmf_user_emb.T,
                   mlp_item_emb.T, gmf_item_emb.T, W1)
    GU, GI = _gather(users.reshape(_NW, _NCH, 128),
                     items.reshape(_NW, _NCH, 128), U, I)
    return _mlp(GU, GI, b1.reshape(1, H1), W2, b2.reshape(1, 1))


# speedup vs baseline: 2.9643x; 1.0484x over previous
import functools

import jax
import jax.numpy as jnp
from jax import lax
from jax.experimental import pallas as pl
from jax.experimental.pallas import tpu as pltpu
from jax.experimental.pallas import tpu_sc as plsc

V = 1000000
BATCH = 16384
MLP_EMB = 64
GMF_EMB = 32
H1 = 64

_NC = 2
_NS = 16
_NW = _NC * _NS
_BPW = BATCH // _NW
_NCH = _BPW // 128
_RBLK = 12800
_RNB = (V + _RBLK - 1) // _RBLK

_DN = (((0,), (0,)), ((), ()))


def _eye(n):
    return (lax.broadcasted_iota(jnp.int32, (n, n), 0)
            == lax.broadcasted_iota(jnp.int32, (n, n), 1)
            ).astype(jnp.bfloat16)


def _repack_body(mu_t_ref, gu_t_ref, mi_t_ref, gi_t_ref, w1_ref,
                 u_out_ref, i_out_ref):
    w1 = w1_ref[...].astype(jnp.bfloat16)
    eye32 = _eye(GMF_EMB)
    for tref, wmat, oref in (
        (mu_t_ref, w1[:MLP_EMB], u_out_ref),
        (mi_t_ref, w1[MLP_EMB:], i_out_ref),
    ):
        oref[:, :MLP_EMB] = lax.dot_general(
            tref[...].astype(jnp.bfloat16), wmat, _DN,
            preferred_element_type=jnp.float32)
    for tref, oref in ((gu_t_ref, u_out_ref), (gi_t_ref, i_out_ref)):
        oref[:, MLP_EMB:MLP_EMB + GMF_EMB] = lax.dot_general(
            tref[...].astype(jnp.bfloat16), eye32, _DN,
            preferred_element_type=jnp.float32)


def _repack(mu_t, gu_t, mi_t, gi_t, W1):
    return pl.pallas_call(
        _repack_body,
        grid=(_RNB,),
        in_specs=[
            pl.BlockSpec((MLP_EMB, _RBLK), lambda i: (0, i)),
            pl.BlockSpec((GMF_EMB, _RBLK), lambda i: (0, i)),
            pl.BlockSpec((MLP_EMB, _RBLK), lambda i: (0, i)),
            pl.BlockSpec((GMF_EMB, _RBLK), lambda i: (0, i)),
            pl.BlockSpec((2 * MLP_EMB, H1), lambda i: (0, 0)),
        ],
        out_specs=[
            pl.BlockSpec((_RBLK, 128), lambda i: (i, 0)),
            pl.BlockSpec((_RBLK, 128), lambda i: (i, 0)),
        ],
        out_shape=[
            jax.ShapeDtypeStruct((V, 128), jnp.float32),
            jax.ShapeDtypeStruct((V, 128), jnp.float32),
        ],
    )(mu_t, gu_t, mi_t, gi_t, W1)


def _gather_body(uidx_hbm, iidx_hbm, u_tab, i_tab, gu_out, gi_out,
                 uidx_v, iidx_v, rows_v, sem):
    wid = lax.axis_index("s") * _NC + lax.axis_index("c")
    base = wid * _BPW
    pltpu.sync_copy(uidx_hbm.at[wid], uidx_v)
    pltpu.sync_copy(iidx_hbm.at[wid], iidx_v)
    for j in range(_NCH):
        pltpu.async_copy(u_tab.at[uidx_v.at[j]],
                         rows_v.at[pl.ds(j * 128, 128)], sem)
    for j in range(_NCH):
        pltpu.make_async_copy(u_tab.at[uidx_v.at[j]],
                              rows_v.at[pl.ds(j * 128, 128)], sem).wait()
    pltpu.sync_copy(rows_v, gu_out.at[pl.ds(base, _BPW)])
    for j in range(_NCH):
        pltpu.async_copy(i_tab.at[iidx_v.at[j]],
                         rows_v.at[pl.ds(j * 128, 128)], sem)
    for j in range(_NCH):
        pltpu.make_async_copy(i_tab.at[iidx_v.at[j]],
                              rows_v.at[pl.ds(j * 128, 128)], sem).wait()
    pltpu.sync_copy(rows_v, gi_out.at[pl.ds(base, _BPW)])


_gather = functools.partial(
    pl.kernel,
    mesh=plsc.VectorSubcoreMesh(core_axis_name="c", subcore_axis_name="s"),
    out_type=[
        jax.ShapeDtypeStruct((BATCH, 128), jnp.float32),
        jax.ShapeDtypeStruct((BATCH, 128), jnp.float32),
    ],
    scratch_types=[
        pltpu.VMEM((_NCH, 128), jnp.int32),
        pltpu.VMEM((_NCH, 128), jnp.int32),
        pltpu.VMEM((_BPW, 128), jnp.float32),
        pltpu.SemaphoreType.DMA,
    ],
    compiler_params=pltpu.CompilerParams(use_tc_tiling_on_sc=True),
)(_gather_body)


def _mlp_body(gu_ref, gi_ref, b1_ref, W2_ref, b2_ref, out_ref):
    gu_all = gu_ref[...]
    gi_all = gi_ref[...]
    h = jnp.maximum(
        gu_all[:, :H1] + gi_all[:, :H1] + b1_ref[...], 0.0)
    gmf = (gu_all[:, MLP_EMB:MLP_EMB + GMF_EMB]
           * gi_all[:, MLP_EMB:MLP_EMB + GMF_EMB])
    w2 = W2_ref[...]
    z = jnp.dot(h, w2[:H1, :], preferred_element_type=jnp.float32)
    z += jnp.dot(gmf, w2[H1:, :], preferred_element_type=jnp.float32)
    out_ref[...] = jax.nn.sigmoid(z + b2_ref[...])


def _mlp(GU, GI, b1, W2, b2, blk=2048):
    nblk = BATCH // blk
    return pl.pallas_call(
        _mlp_body,
        grid=(nblk,),
        in_specs=[
            pl.BlockSpec((blk, 128), lambda i: (i, 0)),
            pl.BlockSpec((blk, 128), lambda i: (i, 0)),
            pl.BlockSpec((1, H1), lambda i: (0, 0)),
            pl.BlockSpec((H1 + GMF_EMB, 1), lambda i: (0, 0)),
            pl.BlockSpec((1, 1), lambda i: (0, 0)),
        ],
        out_specs=pl.BlockSpec((blk, 1), lambda i: (i, 0)),
        out_shape=jax.ShapeDtypeStruct((BATCH, 1), jnp.float32),
    )(GU, GI, b1, W2, b2)


@jax.jit
def kernel(users, items, mlp_user_emb, mlp_item_emb, gmf_user_emb,
           gmf_item_emb, W1, b1, W2, b2):
    U, I = _repack(mlp_user_emb.T, gmf_user_emb.T,
                   mlp_item_emb.T, gmf_item_emb.T, W1)
    GU, GI = _gather(users.reshape(_NW, _NCH, 128),
                     items.reshape(_NW, _NCH, 128), U, I)
    return _mlp(GU, GI, b1.reshape(1, H1), W2, b2.reshape(1, 1))


# speedup vs baseline: 2.9700x; 1.0019x over previous
import functools

import jax
import jax.numpy as jnp
from jax import lax
from jax.experimental import pallas as pl
from jax.experimental.pallas import tpu as pltpu
from jax.experimental.pallas import tpu_sc as plsc

V = 1000000
BATCH = 16384
MLP_EMB = 64
GMF_EMB = 32
H1 = 64

_NC = 2
_NS = 16
_NW = _NC * _NS
_BPW = BATCH // _NW
_NCH = _BPW // 128
_RBLK = 12800
_RNB = (V + _RBLK - 1) // _RBLK

_DN = (((0,), (0,)), ((), ()))


def _eye(n):
    return (lax.broadcasted_iota(jnp.int32, (n, n), 0)
            == lax.broadcasted_iota(jnp.int32, (n, n), 1)
            ).astype(jnp.bfloat16)


def _repack_body(mu_t_ref, gu_t_ref, mi_t_ref, gi_t_ref, w1_ref,
                 u_out_ref, i_out_ref):
    w1 = w1_ref[...].astype(jnp.bfloat16)
    eye32 = _eye(GMF_EMB)
    for tref, wmat, oref in (
        (mu_t_ref, w1[:MLP_EMB], u_out_ref),
        (mi_t_ref, w1[MLP_EMB:], i_out_ref),
    ):
        oref[:, :MLP_EMB] = lax.dot_general(
            tref[...].astype(jnp.bfloat16), wmat, _DN,
            preferred_element_type=jnp.float32)
    for tref, oref in ((gu_t_ref, u_out_ref), (gi_t_ref, i_out_ref)):
        oref[:, MLP_EMB:MLP_EMB + GMF_EMB] = lax.dot_general(
            tref[...].astype(jnp.bfloat16), eye32, _DN,
            preferred_element_type=jnp.float32)


def _repack(mu_t, gu_t, mi_t, gi_t, W1):
    return pl.pallas_call(
        _repack_body,
        grid=(_RNB,),
        in_specs=[
            pl.BlockSpec((MLP_EMB, _RBLK), lambda i: (0, i)),
            pl.BlockSpec((GMF_EMB, _RBLK), lambda i: (0, i)),
            pl.BlockSpec((MLP_EMB, _RBLK), lambda i: (0, i)),
            pl.BlockSpec((GMF_EMB, _RBLK), lambda i: (0, i)),
            pl.BlockSpec((2 * MLP_EMB, H1), lambda i: (0, 0)),
        ],
        out_specs=[
            pl.BlockSpec((_RBLK, 128), lambda i: (i, 0)),
            pl.BlockSpec((_RBLK, 128), lambda i: (i, 0)),
        ],
        out_shape=[
            jax.ShapeDtypeStruct((V, 128), jnp.float32),
            jax.ShapeDtypeStruct((V, 128), jnp.float32),
        ],
        compiler_params=pltpu.CompilerParams(
            vmem_limit_bytes=100 * 1024 * 1024),
    )(mu_t, gu_t, mi_t, gi_t, W1)


def _gather_body(uidx_hbm, iidx_hbm, u_tab, i_tab, gu_out, gi_out,
                 uidx_v, iidx_v, rows_v, sem):
    wid = lax.axis_index("s") * _NC + lax.axis_index("c")
    base = wid * _BPW
    pltpu.sync_copy(uidx_hbm.at[wid], uidx_v)
    pltpu.sync_copy(iidx_hbm.at[wid], iidx_v)
    for j in range(_NCH):
        pltpu.async_copy(u_tab.at[uidx_v.at[j]],
                         rows_v.at[pl.ds(j * 128, 128)], sem)
    for j in range(_NCH):
        pltpu.make_async_copy(u_tab.at[uidx_v.at[j]],
                              rows_v.at[pl.ds(j * 128, 128)], sem).wait()
    pltpu.sync_copy(rows_v, gu_out.at[pl.ds(base, _BPW)])
    for j in range(_NCH):
        pltpu.async_copy(i_tab.at[iidx_v.at[j]],
                         rows_v.at[pl.ds(j * 128, 128)], sem)
    for j in range(_NCH):
        pltpu.make_async_copy(i_tab.at[iidx_v.at[j]],
                              rows_v.at[pl.ds(j * 128, 128)], sem).wait()
    pltpu.sync_copy(rows_v, gi_out.at[pl.ds(base, _BPW)])


_gather = functools.partial(
    pl.kernel,
    mesh=plsc.VectorSubcoreMesh(core_axis_name="c", subcore_axis_name="s"),
    out_type=[
        jax.ShapeDtypeStruct((BATCH, 128), jnp.float32),
        jax.ShapeDtypeStruct((BATCH, 128), jnp.float32),
    ],
    scratch_types=[
        pltpu.VMEM((_NCH, 128), jnp.int32),
        pltpu.VMEM((_NCH, 128), jnp.int32),
        pltpu.VMEM((_BPW, 128), jnp.float32),
        pltpu.SemaphoreType.DMA,
    ],
    compiler_params=pltpu.CompilerParams(use_tc_tiling_on_sc=True),
)(_gather_body)


def _mlp_body(gu_ref, gi_ref, b1_ref, W2_ref, b2_ref, out_ref):
    gu_all = gu_ref[...]
    gi_all = gi_ref[...]
    h = jnp.maximum(
        gu_all[:, :H1] + gi_all[:, :H1] + b1_ref[...], 0.0)
    gmf = (gu_all[:, MLP_EMB:MLP_EMB + GMF_EMB]
           * gi_all[:, MLP_EMB:MLP_EMB + GMF_EMB])
    w2 = W2_ref[...]
    z = jnp.dot(h, w2[:H1, :], preferred_element_type=jnp.float32)
    z += jnp.dot(gmf, w2[H1:, :], preferred_element_type=jnp.float32)
    out_ref[...] = jax.nn.sigmoid(z + b2_ref[...])


def _mlp(GU, GI, b1, W2, b2, blk=2048):
    nblk = BATCH // blk
    return pl.pallas_call(
        _mlp_body,
        grid=(nblk,),
        in_specs=[
            pl.BlockSpec((blk, 128), lambda i: (i, 0)),
            pl.BlockSpec((blk, 128), lambda i: (i, 0)),
            pl.BlockSpec((1, H1), lambda i: (0, 0)),
            pl.BlockSpec((H1 + GMF_EMB, 1), lambda i: (0, 0)),
            pl.BlockSpec((1, 1), lambda i: (0, 0)),
        ],
        out_specs=pl.BlockSpec((blk, 1), lambda i: (i, 0)),
        out_shape=jax.ShapeDtypeStruct((BATCH, 1), jnp.float32),
    )(GU, GI, b1, W2, b2)


@jax.jit
def kernel(users, items, mlp_user_emb, mlp_item_emb, gmf_user_emb,
           gmf_item_emb, W1, b1, W2, b2):
    U, I = _repack(mlp_user_emb.T, gmf_user_emb.T,
                   mlp_item_emb.T, gmf_item_emb.T, W1)
    GU, GI = _gather(users.reshape(_NW, _NCH, 128),
                     items.reshape(_NW, _NCH, 128), U, I)
    return _mlp(GU, GI, b1.reshape(1, H1), W2, b2.reshape(1, 1))
